# baseline (speedup 1.0000x reference)
import functools

import jax
import jax.numpy as jnp
from jax import lax
from jax.experimental import pallas as pl
from jax.experimental.pallas import tpu as pltpu
from jax.experimental.pallas import tpu_sc as plsc

N_NODES = 100000
N_EDGES = 3200000
D_NODE = 128
D_EDGE = 16
D_OUT = 128

NC = 2
NS = 16
NW = NC * NS

CHUNK = 256
NCH = N_EDGES // CHUNK
CH_MAIN = 388
NITER = CH_MAIN // 4
CH_W = 390
CH_LEFT = NCH - CH_W * NW
B = 128
PITCH = CHUNK + 1

N_PAD = 102400
ROWS_S = N_PAD // NS
WSTEPS = ROWS_S // CHUNK


def _sc_segment_sum(ea_t, recv):
    mesh = plsc.VectorSubcoreMesh(core_axis_name="c", subcore_axis_name="s")

    @functools.partial(
        pl.kernel,
        mesh=mesh,
        compiler_params=pltpu.CompilerParams(needs_layout_passes=False,
                                             use_tc_tiling_on_sc=False),
        out_type=jax.ShapeDtypeStruct((NC, N_PAD, D_EDGE), jnp.float32),
        scratch_types=[
            pltpu.VMEM((D_EDGE, PITCH), jnp.float32),
            pltpu.VMEM((D_EDGE, PITCH), jnp.float32),
            pltpu.VMEM((CHUNK, D_EDGE), jnp.float32),
            pltpu.VMEM((CHUNK, D_EDGE), jnp.float32),
            pltpu.VMEM((4, CHUNK // B, B), jnp.int32),
            pltpu.VMEM_SHARED((N_PAD, D_EDGE), jnp.float32),
            pltpu.SemaphoreType.DMA,
            pltpu.SemaphoreType.DMA,
            pltpu.SemaphoreType.DMA,
            pltpu.SemaphoreType.DMA,
        ],
    )
    def k(ea_hbm, recv_hbm, out_hbm, fbuf0, fbuf1, ebuf0, ebuf1, ibuf, acc,
          fsem0, fsem1, ssem0, ssem1):
        c = lax.axis_index("c")
        s = lax.axis_index("s")
        wid = s * NC + c
        lanes = lax.broadcasted_iota(jnp.int32, (D_EDGE,), 0)
        fbuf = (fbuf0, fbuf1)
        ebuf = (ebuf0, ebuf1)
        fsem = (fsem0, fsem1)
        ssem = (ssem0, ssem1)

        lo = wid * CH_W

        def fire_fetch(ch, b, q):
            for f in range(D_EDGE):
                pltpu.async_copy(
                    ea_hbm.at[pl.ds(f * N_EDGES + ch * CHUNK, CHUNK)],
                    fbuf[b].at[f, pl.ds(0, CHUNK)], fsem[b])
            pltpu.async_copy(recv_hbm.at[ch], ibuf.at[q], fsem[b])

        def wait_fetch(b):
            for f in range(D_EDGE):
                pltpu.make_async_copy(ea_hbm.at[pl.ds(0, CHUNK)],
                                      fbuf[b].at[f, pl.ds(0, CHUNK)],
                                      fsem[b]).wait()
            pltpu.make_async_copy(recv_hbm.at[0], ibuf.at[0], fsem[b]).wait()

        def fire_scatter(b, q):
            for r in range(CHUNK // B):
                pltpu.async_copy(ebuf[b].at[pl.ds(r * B, B)],
                                 acc.at[ibuf.at[q, r]], ssem[b], add=True)

        def wait_scatter(b):
            for r in range(CHUNK // B):
                pltpu.make_async_copy(ebuf[b].at[pl.ds(0, B)],
                                      acc.at[ibuf.at[0, 0]], ssem[b]).wait()

        def transpose(b):
            def tr_body(j, carry):
                cols = jnp.full((D_EDGE,), j * D_EDGE, jnp.int32)
                for i in range(D_EDGE):
                    v = plsc.load_gather(fbuf[b], [lanes, cols + i])
                    ebuf[b][j * D_EDGE + i] = v
                return carry

            lax.fori_loop(0, CHUNK // D_EDGE, tr_body, 0)

        fire_fetch(lo, 0, 0)
        fire_fetch(lo + 1, 1, 1)

        def zero_body(i, carry):
            ebuf0[i] = jnp.zeros((D_EDGE,), jnp.float32)
            return carry

        lax.fori_loop(0, CHUNK, zero_body, 0)
        base = s * ROWS_S
        for t in range(WSTEPS):
            pltpu.sync_copy(ebuf0, acc.at[pl.ds(base + t * CHUNK, CHUNK)])
        plsc.subcore_barrier()

        def chunk_body(it, carry):
            ch0 = lo + it * 4
            for q in range(4):
                b = q % 2
                wait_fetch(b)
                if q >= 2:
                    wait_scatter(b)
                else:
                    @pl.when(it > 0)
                    def _():
                        wait_scatter(b)
                transpose(b)
                fire_scatter(b, q)
                fire_fetch(ch0 + q + 2, b, (q + 2) % 4)
            return carry

        lax.fori_loop(0, NITER, chunk_body, 0)

        for q in range(2):
            b = q
            wait_fetch(b)
            wait_scatter(b)
            transpose(b)
            fire_scatter(b, q)
        wait_scatter(0)
        wait_scatter(1)

        @pl.when(wid < CH_LEFT)
        def _():
            ch = NW * CH_W + wid
            for f in range(D_EDGE):
                pltpu.sync_copy(
                    ea_hbm.at[pl.ds(f * N_EDGES + ch * CHUNK, CHUNK)],
                    fbuf0.at[f, pl.ds(0, CHUNK)])
            pltpu.sync_copy(recv_hbm.at[ch], ibuf.at[0])
            transpose(0)
            for r in range(CHUNK // B):
                pltpu.sync_copy(ebuf0.at[pl.ds(r * B, B)],
                                acc.at[ibuf.at[0, r]], add=True)

        plsc.subcore_barrier()

        for t in range(WSTEPS):
            b = t % 2
            if t >= 2:
                pltpu.make_async_copy(ebuf[b],
                                      out_hbm.at[c, pl.ds(0, CHUNK)],
                                      fsem[b]).wait()
            off = base + t * CHUNK
            pltpu.sync_copy(acc.at[pl.ds(off, CHUNK)], ebuf[b])
            pltpu.async_copy(ebuf[b], out_hbm.at[c, pl.ds(off, CHUNK)],
                             fsem[b])
        for b in range(2):
            pltpu.make_async_copy(ebuf[b], out_hbm.at[c, pl.ds(0, CHUNK)],
                                  fsem[b]).wait()

    return k(ea_t, recv)


_R = 1000


def _mm_body(parts_ref, nodes_ref, w_ref, b_ref, out_ref):
    agg = parts_ref[0] + parts_ref[1]
    out_ref[...] = (
        jnp.dot(agg, w_ref[0:D_EDGE, :], preferred_element_type=jnp.float32)
        + jnp.dot(nodes_ref[...], w_ref[D_EDGE:, :],
                  preferred_element_type=jnp.float32)
        + b_ref[...]
    )


def _tc_matmul(parts, nodes, W, b):
    grid = (N_NODES // _R,)
    return pl.pallas_call(
        _mm_body,
        grid=grid,
        in_specs=[
            pl.BlockSpec((NC, _R, D_EDGE), lambda i: (0, i, 0)),
            pl.BlockSpec((_R, D_NODE), lambda i: (i, 0)),
            pl.BlockSpec((D_EDGE + D_NODE, D_OUT), lambda i: (0, 0)),
            pl.BlockSpec((1, D_OUT), lambda i: (0, 0)),
        ],
        out_specs=pl.BlockSpec((_R, D_OUT), lambda i: (i, 0)),
        out_shape=jax.ShapeDtypeStruct((N_NODES, D_OUT), jnp.float32),
    )(parts, nodes, W, b.reshape(1, D_OUT))


def kernel(nodes, edge_attr, senders, receivers, W, b):
    del senders
    ea_t = edge_attr.T.reshape(-1)
    recv = receivers.astype(jnp.int32).reshape(NCH, CHUNK // B, B)
    parts = _sc_segment_sum(ea_t, recv)
    return _tc_matmul(parts, nodes, W, b)


# speedup vs baseline: 1.0351x; 1.0351x over previous
import functools

import jax
import jax.numpy as jnp
from jax import lax
from jax.experimental import pallas as pl
from jax.experimental.pallas import tpu as pltpu
from jax.experimental.pallas import tpu_sc as plsc

N_NODES = 100000
N_EDGES = 3200000
D_NODE = 128
D_EDGE = 16
D_OUT = 128

NC = 2
NS = 16
NW = NC * NS

CHUNK = 256
NCH = N_EDGES // CHUNK
CH_MAIN = 388
NITER = CH_MAIN // 4
CH_W = 390
CH_LEFT = NCH - CH_W * NW
B = 128
PITCH = CHUNK + 1

N_PAD = 102400
ROWS_S = N_PAD // NS
WSTEPS = ROWS_S // CHUNK


def _sc_segment_sum(ea_t, recv):
    mesh = plsc.VectorSubcoreMesh(core_axis_name="c", subcore_axis_name="s")

    @functools.partial(
        pl.kernel,
        mesh=mesh,
        compiler_params=pltpu.CompilerParams(needs_layout_passes=False,
                                             use_tc_tiling_on_sc=False),
        out_type=jax.ShapeDtypeStruct((NC, N_PAD, D_EDGE), jnp.float32),
        scratch_types=[
            pltpu.VMEM((D_EDGE, PITCH), jnp.float32),
            pltpu.VMEM((D_EDGE, PITCH), jnp.float32),
            pltpu.VMEM((CHUNK, D_EDGE), jnp.float32),
            pltpu.VMEM((CHUNK, D_EDGE), jnp.float32),
            pltpu.VMEM((4, CHUNK // B, B), jnp.int32),
            pltpu.VMEM_SHARED((N_PAD, D_EDGE), jnp.float32),
            pltpu.SemaphoreType.DMA,
            pltpu.SemaphoreType.DMA,
            pltpu.SemaphoreType.DMA,
            pltpu.SemaphoreType.DMA,
        ],
    )
    def k(ea_hbm, recv_hbm, out_hbm, fbuf0, fbuf1, ebuf0, ebuf1, ibuf, acc,
          fsem0, fsem1, ssem0, ssem1):
        c = lax.axis_index("c")
        s = lax.axis_index("s")
        wid = s * NC + c
        lanes = lax.broadcasted_iota(jnp.int32, (D_EDGE,), 0)
        fbuf = (fbuf0, fbuf1)
        ebuf = (ebuf0, ebuf1)
        fsem = (fsem0, fsem1)
        ssem = (ssem0, ssem1)

        lo = wid * CH_W

        def fire_fetch(ch, b, q):
            pltpu.async_copy(ea_hbm.at[:, pl.ds(ch * CHUNK, CHUNK)],
                             fbuf[b].at[:, pl.ds(0, CHUNK)], fsem[b])
            pltpu.async_copy(recv_hbm.at[ch], ibuf.at[q], fsem[b])

        def wait_fetch(b):
            pltpu.make_async_copy(ea_hbm.at[:, pl.ds(0, CHUNK)],
                                  fbuf[b].at[:, pl.ds(0, CHUNK)],
                                  fsem[b]).wait()
            pltpu.make_async_copy(recv_hbm.at[0], ibuf.at[0], fsem[b]).wait()

        def fire_scatter(b, q):
            for r in range(CHUNK // B):
                pltpu.async_copy(ebuf[b].at[pl.ds(r * B, B)],
                                 acc.at[ibuf.at[q, r]], ssem[b], add=True)

        def wait_scatter(b):
            for r in range(CHUNK // B):
                pltpu.make_async_copy(ebuf[b].at[pl.ds(0, B)],
                                      acc.at[ibuf.at[0, 0]], ssem[b]).wait()

        def transpose(b):
            def tr_body(j, carry):
                cols = jnp.full((D_EDGE,), j * D_EDGE, jnp.int32)
                for i in range(D_EDGE):
                    v = plsc.load_gather(fbuf[b], [lanes, cols + i])
                    ebuf[b][j * D_EDGE + i] = v
                return carry

            lax.fori_loop(0, CHUNK // D_EDGE, tr_body, 0)

        fire_fetch(lo, 0, 0)
        fire_fetch(lo + 1, 1, 1)

        def zero_body(i, carry):
            ebuf0[i] = jnp.zeros((D_EDGE,), jnp.float32)
            return carry

        lax.fori_loop(0, CHUNK, zero_body, 0)
        base = s * ROWS_S
        for t in range(WSTEPS):
            pltpu.sync_copy(ebuf0, acc.at[pl.ds(base + t * CHUNK, CHUNK)])
        plsc.subcore_barrier()

        def chunk_body(it, carry):
            ch0 = lo + it * 4
            for q in range(4):
                b = q % 2
                wait_fetch(b)
                if q >= 2:
                    wait_scatter(b)
                else:
                    @pl.when(it > 0)
                    def _():
                        wait_scatter(b)
                transpose(b)
                fire_scatter(b, q)
                fire_fetch(ch0 + q + 2, b, (q + 2) % 4)
            return carry

        lax.fori_loop(0, NITER, chunk_body, 0)

        for q in range(2):
            b = q
            wait_fetch(b)
            wait_scatter(b)
            transpose(b)
            fire_scatter(b, q)
        wait_scatter(0)
        wait_scatter(1)

        @pl.when(wid < CH_LEFT)
        def _():
            ch = NW * CH_W + wid
            pltpu.sync_copy(ea_hbm.at[:, pl.ds(ch * CHUNK, CHUNK)],
                            fbuf0.at[:, pl.ds(0, CHUNK)])
            pltpu.sync_copy(recv_hbm.at[ch], ibuf.at[0])
            transpose(0)
            for r in range(CHUNK // B):
                pltpu.sync_copy(ebuf0.at[pl.ds(r * B, B)],
                                acc.at[ibuf.at[0, r]], add=True)

        plsc.subcore_barrier()

        for t in range(WSTEPS):
            b = t % 2
            if t >= 2:
                pltpu.make_async_copy(ebuf[b],
                                      out_hbm.at[c, pl.ds(0, CHUNK)],
                                      fsem[b]).wait()
            off = base + t * CHUNK
            pltpu.sync_copy(acc.at[pl.ds(off, CHUNK)], ebuf[b])
            pltpu.async_copy(ebuf[b], out_hbm.at[c, pl.ds(off, CHUNK)],
                             fsem[b])
        for b in range(2):
            pltpu.make_async_copy(ebuf[b], out_hbm.at[c, pl.ds(0, CHUNK)],
                                  fsem[b]).wait()

    return k(ea_t, recv)


_R = 1000


def _mm1_body(nodes_ref, w_ref, b_ref, out_ref):
    out_ref[...] = (
        jnp.dot(nodes_ref[...], w_ref[...],
                preferred_element_type=jnp.float32)
        + b_ref[...]
    )


def _tc_mm1(nodes, W, b):
    grid = (N_NODES // _R,)
    return pl.pallas_call(
        _mm1_body,
        grid=grid,
        in_specs=[
            pl.BlockSpec((_R, D_NODE), lambda i: (i, 0)),
            pl.BlockSpec((D_NODE, D_OUT), lambda i: (0, 0)),
            pl.BlockSpec((1, D_OUT), lambda i: (0, 0)),
        ],
        out_specs=pl.BlockSpec((_R, D_OUT), lambda i: (i, 0)),
        out_shape=jax.ShapeDtypeStruct((N_NODES, D_OUT), jnp.float32),
    )(nodes, W, b.reshape(1, D_OUT))


def _mm2_body(parts_ref, out1_ref, w_ref, out_ref):
    agg = parts_ref[0] + parts_ref[1]
    out_ref[...] = (
        jnp.dot(agg, w_ref[...], preferred_element_type=jnp.float32)
        + out1_ref[...]
    )


def _tc_mm2(parts, out1, W):
    grid = (N_NODES // _R,)
    return pl.pallas_call(
        _mm2_body,
        grid=grid,
        in_specs=[
            pl.BlockSpec((NC, _R, D_EDGE), lambda i: (0, i, 0)),
            pl.BlockSpec((_R, D_OUT), lambda i: (i, 0)),
            pl.BlockSpec((D_EDGE, D_OUT), lambda i: (0, 0)),
        ],
        out_specs=pl.BlockSpec((_R, D_OUT), lambda i: (i, 0)),
        out_shape=jax.ShapeDtypeStruct((N_NODES, D_OUT), jnp.float32),
    )(parts, out1, W)


def kernel(nodes, edge_attr, senders, receivers, W, b):
    del senders
    ea_t = edge_attr.T
    recv = receivers.astype(jnp.int32).reshape(NCH, CHUNK // B, B)
    out1 = _tc_mm1(nodes, W[D_EDGE:], b)
    parts = _sc_segment_sum(ea_t, recv)
    return _tc_mm2(parts, out1, W[:D_EDGE])
